# speedup vs baseline: 2.8774x; 2.8774x over previous
import functools
import jax
import jax.numpy as jnp
from jax.experimental import pallas as pl

B, E_IN, C, D, K, V = 256, 768, 8, 64, 1024, 256


def _body(batch_ref, rp_ref, wd_ref, bd_ref, cb_ref, val_ref, out_ref):
    x = batch_ref[...]
    rp = rp_ref[0]
    tq = jnp.dot(x, rp, preferred_element_type=jnp.float32)
    y = jnp.dot(tq, wd_ref[0], preferred_element_type=jnp.float32) + bd_ref[0]
    cb = cb_ref[0]
    xe = jax.lax.dot_general(y, cb, (((1,), (1,)), ((), ())),
                             preferred_element_type=jnp.float32)
    x2 = jnp.sum(y * y, axis=1, keepdims=True)
    e2 = jnp.sum(cb * cb, axis=1)
    dist = -(x2 - 2.0 * xe + e2[None, :])
    m = jnp.max(dist, axis=1, keepdims=True)
    kidx = jax.lax.broadcasted_iota(jnp.int32, (B, K), 1)
    idx = jnp.min(jnp.where(dist == m, kidx, K), axis=1, keepdims=True)
    vmean = jnp.mean(val_ref[0], axis=1)
    sel = jnp.where(kidx == idx, vmean[None, :], 0.0)
    out_ref[0, 0, :] = jnp.sum(sel, axis=1)


@jax.jit
def kernel(batch, values, rand_proj, W_in, b_in, codebook):
    W_diag = W_in.reshape(C, D, D).transpose(0, 2, 1)
    b_diag = b_in.reshape(C, 1, D)
    out = pl.pallas_call(
        _body,
        grid=(C,),
        in_specs=[
            pl.BlockSpec((B, E_IN), lambda c: (0, 0)),
            pl.BlockSpec((1, E_IN, D), lambda c: (c, 0, 0)),
            pl.BlockSpec((1, D, D), lambda c: (c, 0, 0)),
            pl.BlockSpec((1, 1, D), lambda c: (c, 0, 0)),
            pl.BlockSpec((1, K, D), lambda c: (c, 0, 0)),
            pl.BlockSpec((1, K, V), lambda c: (c, 0, 0)),
        ],
        out_specs=pl.BlockSpec((1, 1, B), lambda c: (c, 0, 0)),
        out_shape=jax.ShapeDtypeStruct((C, 1, B), jnp.float32),
    )(batch, rand_proj, W_diag, b_diag, codebook, values)
    return out.reshape(C, B).T
